# baseline (speedup 1.0000x reference)
import functools

import jax
import jax.numpy as jnp
from jax import lax
from jax.experimental import pallas as pl
from jax.experimental.pallas import tpu as pltpu
from jax.experimental.pallas import tpu_sc as plsc

N_VOCAB = 100000
EMBED_DIM = 64
PADDED_DIM = 128
OFFSET = 20
BATCH = 4096

_INFO = plsc.get_sparse_core_info()
NC = _INFO.num_cores
NS = _INFO.num_subcores
NW = NC * NS
B_PER_W = BATCH // NW
NB = 16
NSUB = B_PER_W // NB
ROWS = NB * OFFSET
W_IDX = B_PER_W * OFFSET


def _make_kernel():
    mesh = plsc.VectorSubcoreMesh(core_axis_name="c", subcore_axis_name="s")

    @functools.partial(
        pl.kernel,
        mesh=mesh,
        out_type=jax.ShapeDtypeStruct((BATCH, PADDED_DIM), jnp.float32),
        scratch_types=[
            pltpu.VMEM((W_IDX,), jnp.int32),
            pltpu.VMEM((W_IDX,), jnp.int32),
            pltpu.VMEM((2, ROWS, PADDED_DIM), jnp.float32),
            pltpu.VMEM((2, NB, PADDED_DIM), jnp.float32),
            pltpu.SemaphoreType.DMA((2,)),
            pltpu.SemaphoreType.DMA((2,)),
        ],
    )
    def embag(pidx_hbm, off_hbm, table_hbm, out_hbm,
              pidx_v, off_v, rows_v, out_v, gsem, osem):
        wid = lax.axis_index("s") * NC + lax.axis_index("c")
        wbase = wid * B_PER_W
        pltpu.sync_copy(pidx_hbm.at[pl.ds(wbase * OFFSET, W_IDX)], pidx_v)
        pltpu.sync_copy(off_hbm.at[pl.ds(wbase * OFFSET, W_IDX)], off_v)

        def gather(s):
            return pltpu.async_copy(
                table_hbm.at[pidx_v.at[pl.ds(s * ROWS, ROWS)]],
                rows_v.at[s % 2], gsem.at[s % 2])

        gc = {0: gather(0)}
        oc = {}
        for s in range(NSUB):
            if s + 1 < NSUB:
                gc[s + 1] = gather(s + 1)
            gc[s].wait()
            if s >= 2:
                oc[s - 2].wait()

            def body(b, _, buf=s % 2, s=s):
                ro = b * OFFSET
                o0 = off_v[pl.ds(s * ROWS + ro, 16)]
                o1 = off_v[pl.ds(s * ROWS + ro + 4, 16)]
                offs = [o0[j] if j < 16 else o1[j - 4] for j in range(OFFSET)]
                for v in range(EMBED_DIM // 16):
                    acc = rows_v[buf, ro, pl.ds(offs[0] + v * 16, 16)]
                    for j in range(1, OFFSET):
                        acc = acc + rows_v[buf, ro + j,
                                           pl.ds(offs[j] + v * 16, 16)]
                    out_v[buf, b, pl.ds(v * 16, 16)] = acc
                return 0

            lax.fori_loop(0, NB, body, 0)
            oc[s] = pltpu.async_copy(
                out_v.at[s % 2],
                out_hbm.at[pl.ds(wbase + s * NB, NB)], osem.at[s % 2])
        oc[NSUB - 2].wait()
        oc[NSUB - 1].wait()

    return embag


_embag = _make_kernel()


@jax.jit
def kernel(sparse_index_group_batch, sparse_offset_group_batch, weight):
    del sparse_offset_group_batch
    idx = sparse_index_group_batch.astype(jnp.int32)
    pidx = idx >> 1
    off = (idx & 1) * EMBED_DIM
    table = weight.reshape(N_VOCAB // 2, PADDED_DIM)
    return _embag(pidx, off, table)[:, :EMBED_DIM]
